# baseline (speedup 1.0000x reference)
import functools

import jax
import jax.numpy as jnp
from jax import lax
from jax.experimental import pallas as pl
from jax.experimental.pallas import tpu as pltpu
from jax.experimental.pallas import tpu_sc as plsc

N = 10000
NP = 10240
F_IN = 128
H1 = 128
H2 = 64
G = 128
E = 320000
K = 128
CH = 80
NBUF = 2
NPASS = 5
PCH = CH // NPASS
GRP = PCH // NBUF
EPT = K * CH
EP = 32 * EPT
NC = 2
NS = 16

DEG_PT = EP // NS
DEG_ROWS = NP // (NC * NS)
WB = NP // NS


def _mesh():
    return plsc.VectorSubcoreMesh(core_axis_name="c", subcore_axis_name="s")


@functools.partial(
    pl.kernel,
    mesh=_mesh(),
    out_type=jax.ShapeDtypeStruct((NP,), jnp.float32),
    compiler_params=pltpu.CompilerParams(needs_layout_passes=False),
    scratch_types=[
        pltpu.VMEM((2, CH, K), jnp.int32),
        pltpu.VMEM((NP,), jnp.float32),
        pltpu.VMEM((NS * DEG_ROWS,), jnp.float32),
        pltpu.VMEM((DEG_ROWS,), jnp.float32),
        pltpu.VMEM_SHARED((NS * NP,), jnp.float32),
    ],
)
def _deg_kernel(e_hbm, deg_hbm, idxv, hist, buf, degloc, hist2d):
    c = lax.axis_index("c")
    s = lax.axis_index("s")
    pltpu.sync_copy(e_hbm.at[1, pl.ds(2 * s, 2)], idxv)

    def zbody(i, carry):
        hist[pl.ds(i * 16, 16)] = jnp.zeros((16,), jnp.float32)
        return carry

    lax.fori_loop(0, NP // 16, zbody, 0)

    ones = jnp.ones((16,), jnp.float32)

    def sbody(i, carry):
        for p in range(2):
            for u in range(K // 16):
                idx = idxv[p, i, pl.ds(u * 16, 16)]
                plsc.addupdate_scatter(hist, [idx], ones)
        return carry

    lax.fori_loop(0, CH, sbody, 0)

    pltpu.sync_copy(hist, hist2d.at[pl.ds(s * NP, NP)])
    plsc.subcore_barrier()

    base = (c * NS + s) * DEG_ROWS
    for kk in range(NS):
        pltpu.sync_copy(hist2d.at[pl.ds(kk * NP + base, DEG_ROWS)],
                        buf.at[pl.ds(kk * DEG_ROWS, DEG_ROWS)])

    def rbody(j, carry):
        acc = jnp.zeros((16,), jnp.float32)
        for kk in range(NS):
            acc = acc + buf[pl.ds(kk * DEG_ROWS + j * 16, 16)]
        degloc[pl.ds(j * 16, 16)] = acc
        return carry

    lax.fori_loop(0, DEG_ROWS // 16, rbody, 0)
    pltpu.sync_copy(degloc, deg_hbm.at[pl.ds(base, DEG_ROWS)])


def _make_scatter(D):
    @functools.partial(
        pl.kernel,
        mesh=_mesh(),
        out_type=jax.ShapeDtypeStruct((NC, NP, D), jnp.float32),
        compiler_params=pltpu.CompilerParams(needs_layout_passes=False),
        scratch_types=[
            pltpu.VMEM((PCH, K), jnp.int32),
            pltpu.VMEM((PCH, K), jnp.int32),
            pltpu.VMEM((NBUF, K, D), jnp.float32),
            pltpu.VMEM_SHARED((NP, D), jnp.float32),
            pltpu.SemaphoreType.DMA((NBUF,)),
            pltpu.SemaphoreType.DMA((NBUF,)),
        ],
    )
    def _scatter_kernel(y_hbm, e_hbm, out_hbm, srcv, dstv, rows,
                        acc, gsem, ssem):
        c = lax.axis_index("c")
        s = lax.axis_index("s")
        t = c * NS + s

        def zbody(i, carry):
            for kk in range(D // 16):
                rows[0, i, pl.ds(kk * 16, 16)] = jnp.zeros((16,), jnp.float32)
            return carry

        lax.fori_loop(0, K, zbody, 0)
        base = s * WB
        for kk in range(WB // K):
            pltpu.sync_copy(rows.at[0], acc.at[pl.ds(base + kk * K, K)])
        plsc.subcore_barrier()

        for p in range(NPASS):
            pltpu.sync_copy(e_hbm.at[0, t, pl.ds(p * PCH, PCH)], srcv)
            pltpu.sync_copy(e_hbm.at[1, t, pl.ds(p * PCH, PCH)], dstv)
            for b in range(NBUF):
                pltpu.async_copy(y_hbm.at[srcv.at[b]], rows.at[b], gsem.at[b])

            def gbody(g, carry):
                j0 = g * NBUF
                for b in range(NBUF):
                    pltpu.make_async_copy(
                        y_hbm.at[srcv.at[j0 + b]], rows.at[b],
                        gsem.at[b]).wait()
                    pltpu.async_copy(
                        rows.at[b], acc.at[dstv.at[j0 + b]], ssem.at[b],
                        add=True)
                for b in range(NBUF):
                    pltpu.make_async_copy(
                        rows.at[b], acc.at[dstv.at[j0 + b]], ssem.at[b]).wait()

                    @pl.when(g < GRP - 1)
                    def _():
                        pltpu.async_copy(
                            y_hbm.at[srcv.at[j0 + NBUF + b]], rows.at[b],
                            gsem.at[b])
                return carry

            lax.fori_loop(0, GRP, gbody, 0)

        plsc.subcore_barrier()
        pltpu.sync_copy(acc.at[pl.ds(base, WB)], out_hbm.at[c, pl.ds(base, WB)])

    return _scatter_kernel


_scatter128 = _make_scatter(H1)


def _tca0_body(x_ref, w_ref, u_ref):
    u_ref[...] = jnp.dot(x_ref[...], w_ref[...],
                         preferred_element_type=jnp.float32)


def _tca1_body(u_ref, deg_ref, y_ref, d_ref):
    deg = deg_ref[...] + 1.0
    dcol = lax.rsqrt(deg)
    y_ref[:N, :] = u_ref[...] * dcol[:N, :]
    y_ref[N:, :] = jnp.zeros((NP - N, H1), jnp.float32)
    d_ref[...] = dcol


def _tcb_body(agg_ref, y_ref, d_ref, b1_ref, w2_ref, y2_ref):
    aggsum = agg_ref[0] + agg_ref[1] + y_ref[...]
    t = jnp.maximum(aggsum * d_ref[...] + b1_ref[...][None, :], 0.0)
    h2 = jnp.dot(t, w2_ref[...], preferred_element_type=jnp.float32)
    y2_ref[...] = h2 * d_ref[...]


def _tcc_body(agg_ref, y2_ref, d_ref, b2_ref, batch_ref, out_ref):
    osum = agg_ref[0] + agg_ref[1] + y2_ref[...]
    o = osum[:, :H2] * d_ref[...] + b2_ref[...][None, :]
    gid = lax.broadcasted_iota(jnp.int32, (G, 1), 0)
    pmat = (batch_ref[...] == gid).astype(jnp.float32)
    sums = jnp.dot(pmat, o, preferred_element_type=jnp.float32)
    cnts = jnp.sum(pmat, axis=1, keepdims=True)
    out_ref[...] = sums / jnp.maximum(cnts, 1.0)


def _tca0(x, W1):
    return pl.pallas_call(
        _tca0_body,
        out_shape=jax.ShapeDtypeStruct((N, H1), jnp.float32),
    )(x, W1)


def _tca1(u, deg_col):
    return pl.pallas_call(
        _tca1_body,
        out_shape=[
            jax.ShapeDtypeStruct((NP, H1), jnp.float32),
            jax.ShapeDtypeStruct((NP, 1), jnp.float32),
        ],
    )(u, deg_col)


def _tcb(agg1, y1, d_col, b1, W2p):
    return pl.pallas_call(
        _tcb_body,
        out_shape=jax.ShapeDtypeStruct((NP, H1), jnp.float32),
    )(agg1, y1, d_col, b1, W2p)


def _tcc(agg2, y2, d_col, b2, batch_row):
    return pl.pallas_call(
        _tcc_body,
        out_shape=jax.ShapeDtypeStruct((G, H2), jnp.float32),
    )(agg2, y2, d_col, b2, batch_row)


def kernel(x, edge_index, batch, W1, b1, W2, b2):
    jpt = EPT - E // 32
    junk = (N + (jnp.arange(jpt)[None, :] + 8 * jnp.arange(32)[:, None])
            % (NP - N)).astype(jnp.int32)
    ei = edge_index.astype(jnp.int32).reshape(2, 32, E // 32)
    ep = jnp.concatenate(
        [ei, jnp.broadcast_to(junk[None], (2, 32, jpt))],
        axis=2).reshape(2, 32, CH, K)
    batch_row = jnp.pad(batch.astype(jnp.int32), (0, NP - N),
                        constant_values=G).reshape(1, NP)

    W2p = jnp.pad(W2, ((0, 0), (0, H1 - H2)))

    u = _tca0(x, W1)
    deg = _deg_kernel(ep)
    deg_col = deg.reshape(NP, 1)
    y1, d_col = _tca1(u, deg_col)
    agg1 = _scatter128(y1, ep)
    y2 = _tcb(agg1, y1, d_col, b1, W2p)
    agg2 = _scatter128(y2, ep)
    return _tcc(agg2, y2, d_col, b2, batch_row)


# speedup vs baseline: 1.1935x; 1.1935x over previous
import functools

import jax
import jax.numpy as jnp
from jax import lax
from jax.experimental import pallas as pl
from jax.experimental.pallas import tpu as pltpu
from jax.experimental.pallas import tpu_sc as plsc

N = 10000
NP = 10240
F_IN = 128
H1 = 128
H2 = 64
G = 128
E = 320000
K = 80
CH = 128
NBUF = 4
NPASS = 4
PCH = CH // NPASS
GRP = PCH // NBUF
EPT = K * CH
EP = 32 * EPT
NC = 2
NS = 16

DEG_PT = EP // NS
DEG_ROWS = NP // (NC * NS)
WB = NP // NS


def _mesh():
    return plsc.VectorSubcoreMesh(core_axis_name="c", subcore_axis_name="s")


@functools.partial(
    pl.kernel,
    mesh=_mesh(),
    out_type=jax.ShapeDtypeStruct((NP,), jnp.float32),
    compiler_params=pltpu.CompilerParams(needs_layout_passes=False),
    scratch_types=[
        pltpu.VMEM((2, CH, K), jnp.int32),
        pltpu.VMEM((NP,), jnp.float32),
        pltpu.VMEM((NS * DEG_ROWS,), jnp.float32),
        pltpu.VMEM((DEG_ROWS,), jnp.float32),
        pltpu.VMEM_SHARED((NS * NP,), jnp.float32),
    ],
)
def _deg_kernel(e_hbm, deg_hbm, idxv, hist, buf, degloc, hist2d):
    c = lax.axis_index("c")
    s = lax.axis_index("s")
    pltpu.sync_copy(e_hbm.at[1, pl.ds(2 * s, 2)], idxv)

    def zbody(i, carry):
        hist[pl.ds(i * 16, 16)] = jnp.zeros((16,), jnp.float32)
        return carry

    lax.fori_loop(0, NP // 16, zbody, 0)

    ones = jnp.ones((16,), jnp.float32)

    def sbody(i, carry):
        for p in range(2):
            for u in range(K // 16):
                idx = idxv[p, i, pl.ds(u * 16, 16)]
                plsc.addupdate_scatter(hist, [idx], ones)
        return carry

    lax.fori_loop(0, CH, sbody, 0)

    pltpu.sync_copy(hist, hist2d.at[pl.ds(s * NP, NP)])
    plsc.subcore_barrier()

    base = (c * NS + s) * DEG_ROWS
    for kk in range(NS):
        pltpu.sync_copy(hist2d.at[pl.ds(kk * NP + base, DEG_ROWS)],
                        buf.at[pl.ds(kk * DEG_ROWS, DEG_ROWS)])

    def rbody(j, carry):
        acc = jnp.zeros((16,), jnp.float32)
        for kk in range(NS):
            acc = acc + buf[pl.ds(kk * DEG_ROWS + j * 16, 16)]
        degloc[pl.ds(j * 16, 16)] = acc
        return carry

    lax.fori_loop(0, DEG_ROWS // 16, rbody, 0)
    pltpu.sync_copy(degloc, deg_hbm.at[pl.ds(base, DEG_ROWS)])


def _make_scatter(D):
    @functools.partial(
        pl.kernel,
        mesh=_mesh(),
        out_type=jax.ShapeDtypeStruct((NC, NP, D), jnp.float32),
        compiler_params=pltpu.CompilerParams(needs_layout_passes=False),
        scratch_types=[
            pltpu.VMEM((PCH, K), jnp.int32),
            pltpu.VMEM((PCH, K), jnp.int32),
            pltpu.VMEM((NBUF, K, D), jnp.float32),
            pltpu.VMEM_SHARED((NP, D), jnp.float32),
            pltpu.SemaphoreType.DMA((NBUF,)),
            pltpu.SemaphoreType.DMA((NBUF,)),
        ],
    )
    def _scatter_kernel(y_hbm, e_hbm, out_hbm, srcv, dstv, rows,
                        acc, gsem, ssem):
        c = lax.axis_index("c")
        s = lax.axis_index("s")
        t = c * NS + s

        def zbody(i, carry):
            for kk in range(D // 16):
                rows[0, i, pl.ds(kk * 16, 16)] = jnp.zeros((16,), jnp.float32)
            return carry

        lax.fori_loop(0, K, zbody, 0)
        base = s * WB
        for kk in range(WB // K):
            pltpu.sync_copy(rows.at[0], acc.at[pl.ds(base + kk * K, K)])
        plsc.subcore_barrier()

        for p in range(NPASS):
            pltpu.sync_copy(e_hbm.at[0, t, pl.ds(p * PCH, PCH)], srcv)
            pltpu.sync_copy(e_hbm.at[1, t, pl.ds(p * PCH, PCH)], dstv)
            for b in range(NBUF):
                pltpu.async_copy(y_hbm.at[srcv.at[b]], rows.at[b], gsem.at[b])

            def gbody(g, carry):
                j0 = g * NBUF
                for b in range(NBUF):
                    pltpu.make_async_copy(
                        y_hbm.at[srcv.at[j0 + b]], rows.at[b],
                        gsem.at[b]).wait()
                    pltpu.async_copy(
                        rows.at[b], acc.at[dstv.at[j0 + b]], ssem.at[b],
                        add=True)
                for b in range(NBUF):
                    pltpu.make_async_copy(
                        rows.at[b], acc.at[dstv.at[j0 + b]], ssem.at[b]).wait()

                    @pl.when(g < GRP - 1)
                    def _():
                        pltpu.async_copy(
                            y_hbm.at[srcv.at[j0 + NBUF + b]], rows.at[b],
                            gsem.at[b])
                return carry

            lax.fori_loop(0, GRP, gbody, 0)

        plsc.subcore_barrier()
        pltpu.sync_copy(acc.at[pl.ds(base, WB)], out_hbm.at[c, pl.ds(base, WB)])

    return _scatter_kernel


_scatter128 = _make_scatter(H1)


def _tca0_body(x_ref, w_ref, u_ref):
    u_ref[...] = jnp.dot(x_ref[...], w_ref[...],
                         preferred_element_type=jnp.float32)


def _tca1_body(u_ref, deg_ref, y_ref, d_ref):
    deg = deg_ref[...] + 1.0
    dcol = lax.rsqrt(deg)
    y_ref[:N, :] = u_ref[...] * dcol[:N, :]
    y_ref[N:, :] = jnp.zeros((NP - N, H1), jnp.float32)
    d_ref[...] = dcol


def _tcb_body(agg_ref, y_ref, d_ref, b1_ref, w2_ref, y2_ref):
    aggsum = agg_ref[0] + agg_ref[1] + y_ref[...]
    t = jnp.maximum(aggsum * d_ref[...] + b1_ref[...][None, :], 0.0)
    h2 = jnp.dot(t, w2_ref[...], preferred_element_type=jnp.float32)
    y2_ref[...] = h2 * d_ref[...]


def _tcc_body(agg_ref, y2_ref, d_ref, b2_ref, batch_ref, out_ref):
    osum = agg_ref[0] + agg_ref[1] + y2_ref[...]
    o = osum[:, :H2] * d_ref[...] + b2_ref[...][None, :]
    gid = lax.broadcasted_iota(jnp.int32, (G, 1), 0)
    pmat = (batch_ref[...] == gid).astype(jnp.float32)
    sums = jnp.dot(pmat, o, preferred_element_type=jnp.float32)
    cnts = jnp.sum(pmat, axis=1, keepdims=True)
    out_ref[...] = sums / jnp.maximum(cnts, 1.0)


def _tca0(x, W1):
    return pl.pallas_call(
        _tca0_body,
        out_shape=jax.ShapeDtypeStruct((N, H1), jnp.float32),
    )(x, W1)


def _tca1(u, deg_col):
    return pl.pallas_call(
        _tca1_body,
        out_shape=[
            jax.ShapeDtypeStruct((NP, H1), jnp.float32),
            jax.ShapeDtypeStruct((NP, 1), jnp.float32),
        ],
    )(u, deg_col)


def _tcb(agg1, y1, d_col, b1, W2p):
    return pl.pallas_call(
        _tcb_body,
        out_shape=jax.ShapeDtypeStruct((NP, H1), jnp.float32),
    )(agg1, y1, d_col, b1, W2p)


def _tcc(agg2, y2, d_col, b2, batch_row):
    return pl.pallas_call(
        _tcc_body,
        out_shape=jax.ShapeDtypeStruct((G, H2), jnp.float32),
    )(agg2, y2, d_col, b2, batch_row)


def kernel(x, edge_index, batch, W1, b1, W2, b2):
    jpt = EPT - E // 32
    junk = (N + (jnp.arange(jpt)[None, :] + 8 * jnp.arange(32)[:, None])
            % (NP - N)).astype(jnp.int32)
    ei = edge_index.astype(jnp.int32).reshape(2, 32, E // 32)
    ep = jnp.concatenate(
        [ei, jnp.broadcast_to(junk[None], (2, 32, jpt))],
        axis=2).reshape(2, 32, CH, K)
    batch_row = jnp.pad(batch.astype(jnp.int32), (0, NP - N),
                        constant_values=G).reshape(1, NP)

    W2p = jnp.pad(W2, ((0, 0), (0, H1 - H2)))

    u = _tca0(x, W1)
    deg = _deg_kernel(ep)
    deg_col = deg.reshape(NP, 1)
    y1, d_col = _tca1(u, deg_col)
    agg1 = _scatter128(y1, ep)
    y2 = _tcb(agg1, y1, d_col, b1, W2p)
    agg2 = _scatter128(y2, ep)
    return _tcc(agg2, y2, d_col, b2, batch_row)


# speedup vs baseline: 1.2209x; 1.0230x over previous
import functools

import jax
import jax.numpy as jnp
from jax import lax
from jax.experimental import pallas as pl
from jax.experimental.pallas import tpu as pltpu
from jax.experimental.pallas import tpu_sc as plsc

N = 10000
NP = 10240
F_IN = 128
H1 = 128
H2 = 64
G = 128
E = 320000
K = 80
CH = 128
NBUF = 4
NPASS = 4
PCH = CH // NPASS
GRP = PCH // NBUF
EPT = K * CH
EP = 32 * EPT
NC = 2
NS = 16

DEG_ROWS = NP // (NC * NS)
WB = NP // NS


def _mesh():
    return plsc.VectorSubcoreMesh(core_axis_name="c", subcore_axis_name="s")


@functools.partial(
    pl.kernel,
    mesh=_mesh(),
    out_type=jax.ShapeDtypeStruct((NP,), jnp.float32),
    compiler_params=pltpu.CompilerParams(needs_layout_passes=False),
    scratch_types=[
        pltpu.VMEM((2, CH, K), jnp.int32),
        pltpu.VMEM((NP,), jnp.float32),
        pltpu.VMEM((NS * DEG_ROWS,), jnp.float32),
        pltpu.VMEM((DEG_ROWS,), jnp.float32),
        pltpu.VMEM_SHARED((NS * NP,), jnp.float32),
    ],
)
def _deg_kernel(e_hbm, deg_hbm, idxv, hist, buf, degloc, hist2d):
    c = lax.axis_index("c")
    s = lax.axis_index("s")
    pltpu.sync_copy(e_hbm.at[1, pl.ds(2 * s, 2)], idxv)

    def zbody(i, carry):
        hist[pl.ds(i * 16, 16)] = jnp.zeros((16,), jnp.float32)
        return carry

    lax.fori_loop(0, NP // 16, zbody, 0)

    ones = jnp.ones((16,), jnp.float32)

    def sbody(i, carry):
        for p in range(2):
            for u in range(K // 16):
                idx = idxv[p, i, pl.ds(u * 16, 16)]
                plsc.addupdate_scatter(hist, [idx], ones)
        return carry

    lax.fori_loop(0, CH, sbody, 0)

    pltpu.sync_copy(hist, hist2d.at[pl.ds(s * NP, NP)])
    plsc.subcore_barrier()

    base = (c * NS + s) * DEG_ROWS
    for kk in range(NS):
        pltpu.sync_copy(hist2d.at[pl.ds(kk * NP + base, DEG_ROWS)],
                        buf.at[pl.ds(kk * DEG_ROWS, DEG_ROWS)])

    def rbody(j, carry):
        acc = jnp.zeros((16,), jnp.float32)
        for kk in range(NS):
            acc = acc + buf[pl.ds(kk * DEG_ROWS + j * 16, 16)]
        degloc[pl.ds(j * 16, 16)] = acc
        return carry

    lax.fori_loop(0, DEG_ROWS // 16, rbody, 0)
    pltpu.sync_copy(degloc, deg_hbm.at[pl.ds(base, DEG_ROWS)])


def _make_scatter(D):
    @functools.partial(
        pl.kernel,
        mesh=_mesh(),
        out_type=jax.ShapeDtypeStruct((NC, NP, D), jnp.float32),
        compiler_params=pltpu.CompilerParams(needs_layout_passes=False),
        scratch_types=[
            pltpu.VMEM((2, PCH, K), jnp.int32),
            pltpu.VMEM((NBUF, K, D), jnp.float32),
            pltpu.VMEM_SHARED((NP, D), jnp.float32),
            pltpu.SemaphoreType.DMA((NBUF,)),
            pltpu.SemaphoreType.DMA((NBUF,)),
        ],
    )
    def _scatter_kernel(y_hbm, e_hbm, out_hbm, idx2, rows,
                        acc, gsem, ssem):
        c = lax.axis_index("c")
        s = lax.axis_index("s")
        t = c * NS + s

        def zbody(i, carry):
            for kk in range(D // 16):
                rows[0, i, pl.ds(kk * 16, 16)] = jnp.zeros((16,), jnp.float32)
            return carry

        lax.fori_loop(0, K, zbody, 0)
        base = s * WB
        for kk in range(WB // K):
            pltpu.sync_copy(rows.at[0], acc.at[pl.ds(base + kk * K, K)])
        plsc.subcore_barrier()

        for p in range(NPASS):
            pltpu.sync_copy(e_hbm.at[:, t, pl.ds(p * PCH, PCH)], idx2)
            for b in range(NBUF):
                pltpu.async_copy(y_hbm.at[idx2.at[0, b]], rows.at[b],
                                 gsem.at[b])

            def gbody(g, carry):
                j0 = g * NBUF
                for b in range(NBUF):
                    pltpu.make_async_copy(
                        y_hbm.at[idx2.at[0, j0 + b]], rows.at[b],
                        gsem.at[b]).wait()
                    pltpu.async_copy(
                        rows.at[b], acc.at[idx2.at[1, j0 + b]], ssem.at[b],
                        add=True)
                for b in range(NBUF):
                    pltpu.make_async_copy(
                        rows.at[b], acc.at[idx2.at[1, j0 + b]],
                        ssem.at[b]).wait()

                    @pl.when(g < GRP - 1)
                    def _():
                        pltpu.async_copy(
                            y_hbm.at[idx2.at[0, j0 + NBUF + b]], rows.at[b],
                            gsem.at[b])
                return carry

            lax.fori_loop(0, GRP, gbody, 0)

        plsc.subcore_barrier()
        pltpu.sync_copy(acc.at[pl.ds(base, WB)], out_hbm.at[c, pl.ds(base, WB)])

    return _scatter_kernel


_scatter128 = _make_scatter(H1)


def _tca0_body(x_ref, w_ref, u_ref):
    u_ref[...] = jnp.dot(x_ref[...], w_ref[...],
                         preferred_element_type=jnp.float32)


def _tca1_body(u_ref, deg_ref, y_ref, d_ref):
    deg = deg_ref[...] + 1.0
    dcol = lax.rsqrt(deg)
    y_ref[:N, :] = u_ref[...] * dcol[:N, :]
    y_ref[N:, :] = jnp.zeros((NP - N, H1), jnp.float32)
    d_ref[...] = dcol


def _tcb_body(agg_ref, y_ref, d_ref, b1_ref, w2_ref, y2_ref):
    aggsum = agg_ref[0] + agg_ref[1] + y_ref[...]
    t = jnp.maximum(aggsum * d_ref[...] + b1_ref[...][None, :], 0.0)
    h2 = jnp.dot(t, w2_ref[...], preferred_element_type=jnp.float32)
    y2_ref[...] = h2 * d_ref[...]


def _tcc_body(agg_ref, y2_ref, d_ref, b2_ref, batch_ref, out_ref):
    osum = agg_ref[0] + agg_ref[1] + y2_ref[...]
    o = osum[:, :H2] * d_ref[...] + b2_ref[...][None, :]
    gid = lax.broadcasted_iota(jnp.int32, (G, 1), 0)
    pmat = (batch_ref[...] == gid).astype(jnp.float32)
    sums = jnp.dot(pmat, o, preferred_element_type=jnp.float32)
    cnts = jnp.sum(pmat, axis=1, keepdims=True)
    out_ref[...] = sums / jnp.maximum(cnts, 1.0)


def _tca0(x, W1):
    return pl.pallas_call(
        _tca0_body,
        out_shape=jax.ShapeDtypeStruct((N, H1), jnp.float32),
    )(x, W1)


def _tca1(u, deg_col):
    return pl.pallas_call(
        _tca1_body,
        out_shape=[
            jax.ShapeDtypeStruct((NP, H1), jnp.float32),
            jax.ShapeDtypeStruct((NP, 1), jnp.float32),
        ],
    )(u, deg_col)


def _tcb(agg1, y1, d_col, b1, W2p):
    return pl.pallas_call(
        _tcb_body,
        out_shape=jax.ShapeDtypeStruct((NP, H1), jnp.float32),
    )(agg1, y1, d_col, b1, W2p)


def _tcc(agg2, y2, d_col, b2, batch_row):
    return pl.pallas_call(
        _tcc_body,
        out_shape=jax.ShapeDtypeStruct((G, H2), jnp.float32),
    )(agg2, y2, d_col, b2, batch_row)


def kernel(x, edge_index, batch, W1, b1, W2, b2):
    jpt = EPT - E // 32
    junk = (N + (jnp.arange(jpt)[None, :] + 8 * jnp.arange(32)[:, None])
            % (NP - N)).astype(jnp.int32)
    ei = edge_index.astype(jnp.int32).reshape(2, 32, E // 32)
    ep = jnp.concatenate(
        [ei, jnp.broadcast_to(junk[None], (2, 32, jpt))],
        axis=2).reshape(2, 32, CH, K)
    batch_row = jnp.pad(batch.astype(jnp.int32), (0, NP - N),
                        constant_values=G).reshape(1, NP)

    W2p = jnp.pad(W2, ((0, 0), (0, H1 - H2)))

    u = _tca0(x, W1)
    deg = _deg_kernel(ep)
    deg_col = deg.reshape(NP, 1)
    y1, d_col = _tca1(u, deg_col)
    agg1 = _scatter128(y1, ep)
    y2 = _tcb(agg1, y1, d_col, b1, W2p)
    agg2 = _scatter128(y2, ep)
    return _tcc(agg2, y2, d_col, b2, batch_row)
